# baseline (speedup 1.0000x reference)
import functools

import jax
import jax.numpy as jnp
from jax import lax
from jax.experimental import pallas as pl
from jax.experimental.pallas import tpu as pltpu
from jax.experimental.pallas import tpu_sc as plsc

K = 8
B = 16
S = 8192
C = 128
NC = 2
LANES = 16
N_TEC = 32
CHUNK = 512
NCHUNK = S // CHUNK
WIN = 8
NWIN = CHUNK // WIN

_NET8 = (
    (0, 1), (2, 3), (4, 5), (6, 7),
    (0, 2), (1, 3), (4, 6), (5, 7),
    (1, 2), (5, 6), (0, 4), (3, 7),
    (1, 5), (2, 6),
    (1, 4), (3, 6),
    (2, 4), (3, 5),
    (3, 4),
)
_BITONIC8 = (
    (0, 4), (1, 5), (2, 6), (3, 7),
    (0, 2), (1, 3), (4, 6), (5, 7),
    (0, 1), (2, 3), (4, 5), (6, 7),
)


def _ce(b, i, j):
    hi = jnp.maximum(b[i], b[j])
    lo = jnp.minimum(b[i], b[j])
    b[i] = hi
    b[j] = lo


def _merge_window(wb, st):
    for (i, j) in _NET8:
        _ce(wb, i, j)
    ts = [jnp.maximum(st[i], wb[K - 1 - i]) for i in range(K)]
    for (i, j) in _BITONIC8:
        _ce(ts, i, j)
    return tuple(ts)


_mesh = plsc.VectorSubcoreMesh(core_axis_name="c", subcore_axis_name="s")


def _make_sc(bs):
    ch_div = N_TEC // bs
    ch_w = C // ch_div
    ng = ch_w // LANES

    @functools.partial(
        pl.kernel,
        out_type=jax.ShapeDtypeStruct((bs, C * K), jnp.float32),
        mesh=_mesh,
        scratch_types=[
            pltpu.VMEM((CHUNK, ch_w), jnp.float32),
            pltpu.VMEM((CHUNK, ch_w), jnp.float32),
            pltpu.VMEM((ch_w * K,), jnp.float32),
            pltpu.SemaphoreType.DMA,
            pltpu.SemaphoreType.DMA,
        ],
        compiler_params=pltpu.CompilerParams(
            use_tc_tiling_on_sc=False, needs_layout_passes=False
        ),
    )
    def _topk_sc(x_hbm, out_hbm, buf0, buf1, obuf, sem0, sem1):
        wid = lax.axis_index("s") * NC + lax.axis_index("c")
        b = wid // ch_div
        ch0 = (wid % ch_div) * ch_w

        neg = jnp.full((LANES,), -jnp.inf, dtype=jnp.float32)
        states = tuple(tuple(neg for _ in range(K)) for _ in range(ng))

        bufs = (buf0, buf1)
        sems = (sem0, sem1)
        copies = [None, None]

        def start(i):
            copies[i % 2] = pltpu.async_copy(
                x_hbm.at[b, pl.ds(i * CHUNK, CHUNK), pl.ds(ch0, ch_w)],
                bufs[i % 2],
                sems[i % 2],
            )

        start(0)
        for chunk in range(NCHUNK):
            copies[chunk % 2].wait()
            if chunk + 1 < NCHUNK:
                start(chunk + 1)
            buf = bufs[chunk % 2]

            new_states = []
            for half in range(0, ng, 2):
                def body(w, st, buf=buf, half=half):
                    st = tuple(st)
                    for ww in range(2):
                        out_st = []
                        for gg in range(2):
                            g = half + gg
                            wb = [
                                buf[
                                    (w * 2 + ww) * WIN + r,
                                    pl.ds(g * LANES, LANES),
                                ]
                                for r in range(WIN)
                            ]
                            out_st.append(_merge_window(wb, st[gg]))
                        st = tuple(out_st)
                    return st

                pair = (states[half], states[half + 1])
                pair = lax.fori_loop(0, NWIN // 2, body, pair)
                new_states.extend(pair)
            states = tuple(new_states)

        lane = lax.iota(jnp.int32, LANES)
        for g in range(ng):
            for j in range(K):
                idx = lane * K + (g * LANES * K + j)
                plsc.store_scatter(obuf, [idx], states[g][j])
        pltpu.sync_copy(obuf, out_hbm.at[b, pl.ds(ch0 * K, ch_w * K)])

    return _topk_sc


_topk_sc_all = _make_sc(B)


def kernel(inputs):
    x = inputs.reshape(B, S, C)
    return _topk_sc_all(x)


# speedup vs baseline: 1.0616x; 1.0616x over previous
import functools

import jax
import jax.numpy as jnp
from jax import lax
from jax.experimental import pallas as pl
from jax.experimental.pallas import tpu as pltpu
from jax.experimental.pallas import tpu_sc as plsc

K = 8
B = 16
S = 8192
C = 128
NC = 2
LANES = 16
N_TEC = 32
CHUNK = 512
NCHUNK = S // CHUNK
WIN = 8
NWIN = CHUNK // WIN

_NET8 = (
    (0, 1), (2, 3), (4, 5), (6, 7),
    (0, 2), (1, 3), (4, 6), (5, 7),
    (1, 2), (5, 6), (0, 4), (3, 7),
    (1, 5), (2, 6),
    (1, 4), (3, 6),
    (2, 4), (3, 5),
    (3, 4),
)
_BITONIC8 = (
    (0, 4), (1, 5), (2, 6), (3, 7),
    (0, 2), (1, 3), (4, 6), (5, 7),
    (0, 1), (2, 3), (4, 5), (6, 7),
)


def _ce(b, i, j):
    hi = jnp.maximum(b[i], b[j])
    lo = jnp.minimum(b[i], b[j])
    b[i] = hi
    b[j] = lo


def _merge_window(wb, st):
    for (i, j) in _NET8:
        _ce(wb, i, j)
    ts = [jnp.maximum(st[i], wb[K - 1 - i]) for i in range(K)]
    for (i, j) in _BITONIC8:
        _ce(ts, i, j)
    return tuple(ts)


_mesh = plsc.VectorSubcoreMesh(core_axis_name="c", subcore_axis_name="s")


def _make_sc(bs):
    ch_div = N_TEC // bs
    ch_w = C // ch_div
    ng = ch_w // LANES

    @functools.partial(
        pl.kernel,
        out_type=jax.ShapeDtypeStruct((bs, C * K), jnp.float32),
        mesh=_mesh,
        scratch_types=[
            pltpu.VMEM((CHUNK, ch_w), jnp.float32),
            pltpu.VMEM((CHUNK, ch_w), jnp.float32),
            pltpu.VMEM((ch_w * K,), jnp.float32),
            pltpu.SemaphoreType.DMA,
            pltpu.SemaphoreType.DMA,
        ],
        compiler_params=pltpu.CompilerParams(
            use_tc_tiling_on_sc=False, needs_layout_passes=False
        ),
    )
    def _topk_sc(x_hbm, out_hbm, buf0, buf1, obuf, sem0, sem1):
        wid = lax.axis_index("s") * NC + lax.axis_index("c")
        b = wid // ch_div
        ch0 = (wid % ch_div) * ch_w

        neg = jnp.full((LANES,), -jnp.inf, dtype=jnp.float32)
        states = tuple(tuple(neg for _ in range(K)) for _ in range(ng))

        bufs = (buf0, buf1)
        sems = (sem0, sem1)
        copies = [None, None]

        def start(i):
            copies[i % 2] = pltpu.async_copy(
                x_hbm.at[b, pl.ds(i * CHUNK, CHUNK), pl.ds(ch0, ch_w)],
                bufs[i % 2],
                sems[i % 2],
            )

        start(0)
        for chunk in range(NCHUNK):
            copies[chunk % 2].wait()
            if chunk + 1 < NCHUNK:
                start(chunk + 1)
            buf = bufs[chunk % 2]

            new_states = []
            for half in range(0, ng, 2):
                def body(w, st, buf=buf, half=half):
                    out_st = []
                    for gg in range(2):
                        g = half + gg
                        wb = [
                            buf[w * WIN + r, pl.ds(g * LANES, LANES)]
                            for r in range(WIN)
                        ]
                        out_st.append(_merge_window(wb, st[gg]))
                    return tuple(out_st)

                pair = (states[half], states[half + 1])
                pair = lax.fori_loop(0, NWIN, body, pair)
                new_states.extend(pair)
            states = tuple(new_states)

        lane = lax.iota(jnp.int32, LANES)
        for g in range(ng):
            for j in range(K):
                idx = lane * K + (g * LANES * K + j)
                plsc.store_scatter(obuf, [idx], states[g][j])
        pltpu.sync_copy(obuf, out_hbm.at[b, pl.ds(ch0 * K, ch_w * K)])

    return _topk_sc


_topk_sc_all = _make_sc(B)


def kernel(inputs):
    x = inputs.reshape(B, S, C)
    return _topk_sc_all(x)
